# baseline (speedup 1.0000x reference)
import functools
import math

import jax
import jax.numpy as jnp
from jax import lax
from jax.experimental import pallas as pl
from jax.experimental.pallas import tpu as pltpu
from jax.experimental.pallas import tpu_sc as plsc

B = 4
SEQ = 2048
D_F = 1024
SCALE = math.sqrt(D_F)

NC = 2
NS = 16
NW = NC * NS
S_PER_W = SEQ // NW
SUB = 32
N_SUB = S_PER_W // SUB
LANES = 16
D_VECS = D_F // LANES


def _sc_body(x_hbm, idx_hbm, pe_hbm, seg_hbm, out_hbm,
             idx_v, seg_v, pe_v, x_v, o_v, sem):
    wid = lax.axis_index("s") * NC + lax.axis_index("c")
    s0 = wid * S_PER_W

    pltpu.sync_copy(idx_hbm, idx_v)
    pltpu.async_copy(seg_hbm.at[idx_v], seg_v, sem).wait()

    def sub_stripe(h, _):
        s_base = s0 + h * SUB
        pltpu.sync_copy(pe_hbm.at[0, pl.ds(s_base, SUB)], pe_v)

        def batch_iter(b, _):
            pltpu.sync_copy(x_hbm.at[b, pl.ds(s_base, SUB)], x_v)

            def row(r, _):
                for j in range(D_VECS):
                    sl = pl.ds(j * LANES, LANES)
                    o_v[r, sl] = (x_v[r, sl] * SCALE
                                  + pe_v[r, sl] + seg_v[0, sl])
                return 0

            lax.fori_loop(0, SUB, row, 0, unroll=False)
            pltpu.sync_copy(o_v, out_hbm.at[b, pl.ds(s_base, SUB)])
            return 0

        lax.fori_loop(0, B, batch_iter, 0, unroll=False)
        return 0

    lax.fori_loop(0, N_SUB, sub_stripe, 0, unroll=False)


@jax.jit
def _pos_encode_sc(x, seg_idx, pe, seg_table):
    mesh = plsc.VectorSubcoreMesh(core_axis_name="c", subcore_axis_name="s")
    kfn = functools.partial(
        pl.kernel,
        mesh=mesh,
        out_type=jax.ShapeDtypeStruct((B, SEQ, D_F), jnp.float32),
        scratch_types=[
            pltpu.VMEM((8,), jnp.int32),
            pltpu.VMEM((8, D_F), jnp.float32),
            pltpu.VMEM((SUB, D_F), jnp.float32),
            pltpu.VMEM((SUB, D_F), jnp.float32),
            pltpu.VMEM((SUB, D_F), jnp.float32),
            pltpu.SemaphoreType.DMA,
        ],
    )(_sc_body)
    return kfn(x, seg_idx, pe, seg_table)


def kernel(x, view_idx, pe, seg_table):
    seq_len = x.shape[1]
    seg_idx = jnp.full((8,), view_idx * seq_len, dtype=jnp.int32)
    return _pos_encode_sc(x, seg_idx, pe, seg_table)
